# Initial kernel scaffold; baseline (speedup 1.0000x reference)
import jax
import jax.numpy as jnp
from jax.experimental import pallas as pl


def kernel(x, positions, positions_count):
    raise NotImplementedError("write your pallas kernel here")


# speedup vs baseline: 3.8901x; 3.8901x over previous
import math

import jax
import jax.numpy as jnp
from jax import lax
from jax.experimental import pallas as pl
from jax.experimental.pallas import tpu as pltpu
from jax.experimental.pallas import tpu_sc as plsc

BS = 64
N = 32768
D = 16
QUANTILE = 0.9
QI = math.ceil(QUANTILE * (N - 1))
K = N - QI
INB = 3280
NBAD = N - K
COMBO = INB + NBAD + 29

NC = 2
NS = 16
NW = NC * NS
ROWS_PER_W = BS // NW

NVREG = N // 16
CH = 128
RING = 4

LEVELS = ((21, 2048), (10, 2048), (0, 1024))

GOOD_FULL = K // CH
GOOD_TAIL = K - GOOD_FULL * CH
BAD_FULL = NBAD // CH
BAD_TAIL = NBAD - BAD_FULL * CH


def _key16(xv):
    u = plsc.bitcast(xv, jnp.uint32)
    neg = u >= jnp.uint32(0x80000000)
    return jnp.where(neg, ~u, u ^ jnp.uint32(0x80000000))


def _scan_level(ord_ref, nb, kneed):
    nchunks = nb // 16
    lanes = lax.iota(jnp.int32, 16)
    big = jnp.int32(2**30)

    def body(t, carry):
        acc, bmax, smax, amin = carry
        jj = nchunks - 1 - t
        col = ord_ref[pl.ds(0 * nb + 16 * jj, 16)]
        for l in range(1, 16):
            col = col + ord_ref[pl.ds(l * nb + 16 * jj, 16)]
        rev = lax.rev(col, (0,))
        cs = plsc.cumsum(rev)
        a_vec = acc + cs
        bidx = (16 * jj + 15) - lanes
        sel = a_vec >= kneed
        bmax = jnp.maximum(bmax, jnp.max(jnp.where(sel, bidx, -1)))
        smax = jnp.maximum(smax, jnp.max(jnp.where(sel, jnp.int32(-1), a_vec)))
        amin = jnp.minimum(amin, jnp.min(jnp.where(sel, a_vec, big)))
        acc = jnp.max(a_vec)
        return acc, bmax, smax, amin

    acc, bmax, smax, amin = lax.fori_loop(
        0, nchunks,
        body,
        (jnp.int32(0), jnp.int32(-1), jnp.int32(-1), big),
    )
    del acc
    return bmax, jnp.maximum(smax, 0), amin


def _body(x_hbm, pos_hbm, vals_hbm, good_hbm, bad_hbm, cnt_hbm,
          xrow, combo, staging, valbuf, ctmp, bounce, ubuf,
          tailg, tailb, tgstage, tbstage, countacc, gsem, osem):
    cid = lax.axis_index("c")
    sid = lax.axis_index("s")
    wid = sid * NC + cid
    lanes = lax.iota(jnp.int32, 16)
    ones_i = jnp.ones((16,), jnp.int32)

    for ref, base, tailn in ((tailg, GOOD_FULL * CH, GOOD_TAIL),
                             (tailb, BAD_FULL * CH, BAD_TAIL)):
        for off in list(range(0, tailn - 16, 16)) + [tailn - 16]:
            ref[0, pl.ds(off, 16)] = base + off + lanes

    @pl.when(sid == 0)
    def _zero_acc():
        def zbody(i, _):
            xrow[pl.ds(16 * i, 16)] = jnp.zeros((16,), jnp.float32)
            return 0
        lax.fori_loop(0, NVREG, zbody, 0)
        pltpu.sync_copy(xrow, countacc)

    plsc.subcore_barrier()

    def gather_region(row, ord_off, nfull, tailn, out_ref, tail_ref, tstage):
        def wait_g(slot):
            pltpu.make_async_copy(pos_hbm.at[pl.ds(0, CH)], staging.at[slot],
                                  gsem.at[slot]).wait()

        def wait_o(slot):
            pltpu.make_async_copy(staging.at[slot],
                                  out_ref.at[row, pl.ds(0, CH)],
                                  osem.at[slot]).wait()

        def body(g, _):
            slot = lax.rem(g, RING)

            @pl.when(g >= RING)
            def _drain():
                wait_o(slot)

            idx = combo.at[pl.ds(ord_off + CH * g, CH)]
            pltpu.async_copy(pos_hbm.at[idx], staging.at[slot], gsem.at[slot])

            @pl.when(g >= 1)
            def _flush_prev():
                pslot = lax.rem(g - 1, RING)
                wait_g(pslot)
                pltpu.async_copy(staging.at[pslot],
                                 out_ref.at[row, pl.ds(CH * (g - 1), CH)],
                                 osem.at[pslot])

            return 0

        lax.fori_loop(0, nfull, body, 0)
        last = nfull - 1
        lslot = last % RING
        wait_g(lslot)
        pltpu.async_copy(staging.at[lslot],
                         out_ref.at[row, pl.ds(CH * last, CH)],
                         osem.at[lslot])
        tslot = nfull % RING
        wait_o(tslot)
        tpad = (tailn + 7) & ~7
        tidx = combo.at[pl.ds(ord_off + CH * nfull, tpad)]
        pltpu.sync_copy(pos_hbm.at[tidx], staging.at[tslot, pl.ds(0, tpad)])

        def tcopy(r, _):
            tstage[r, pl.ds(0, 16)] = staging[tslot, r, pl.ds(0, 16)]
            return 0
        lax.fori_loop(0, tailn, tcopy, 0)
        pltpu.sync_copy(tstage, out_ref.at[row].at[tail_ref.at[0]])
        for s in range(RING):
            if s != tslot:
                pltpu.make_async_copy(staging.at[s],
                                      out_ref.at[row, pl.ds(0, CH)],
                                      osem.at[s]).wait()

    def count_region(base, limit, ntrips):
        def cbody(cix, _):
            for v in range(8):
                pos = 128 * cix + 16 * v
                raw = combo[pl.ds(base + pos, 16)]
                inr = (pos + lanes) < limit
                bounce[0, pl.ds(16 * v, 16)] = jnp.where(inr, raw, 0)
                ubuf[0, pl.ds(16 * v, 16)] = jnp.where(
                    inr, jnp.float32(1.0), jnp.float32(0.0))
            pltpu.sync_copy(ubuf.at[0], countacc.at[bounce.at[0]], add=True)
            return 0

        lax.fori_loop(0, ntrips, cbody, 0)

    for ridx in range(ROWS_PER_W):
        row = ROWS_PER_W * wid + ridx

        pltpu.sync_copy(x_hbm.at[row], xrow)

        kneed = jnp.int32(K)
        prefix = jnp.uint32(0)
        above = jnp.int32(0)
        t_total = jnp.int32(0)
        pshift = 32
        for (shift, nb) in LEVELS:
            nbits = pshift - shift

            def zb(i, _, nb=nb):
                combo[pl.ds(16 * i, 16)] = jnp.zeros((16,), jnp.int32)
                return 0
            lax.fori_loop(0, nb, zb, 0)

            def hb(i, _, shift=shift, nb=nb, pshift=pshift, prefix=prefix):
                xv = xrow[pl.ds(16 * i, 16)]
                key = _key16(xv)
                bucket = lax.convert_element_type(
                    (key >> shift) & jnp.uint32(nb - 1), jnp.int32)
                flat = (lanes * nb) + bucket
                if pshift == 32:
                    plsc.addupdate_scatter(combo, [flat], ones_i)
                else:
                    match = (key >> pshift) == prefix
                    plsc.addupdate_scatter(combo, [flat], ones_i, mask=match)
                return 0
            lax.fori_loop(0, NVREG, hb, 0)

            b_l, s_l, a_l = _scan_level(combo, nb, kneed)
            t_total = above + a_l
            kneed = kneed - s_l
            above = above + s_l
            prefix = (prefix << nbits) | lax.convert_element_type(
                b_l, jnp.uint32)
            pshift = shift

        q_key = prefix

        ib_base = jnp.int32(INB) + (t_total - K)

        def pb(i, carry):
            p_a, p_i = carry
            xv = xrow[pl.ds(16 * i, 16)]
            key = _key16(xv)
            m = key >= q_key
            mi = jnp.where(m, 1, 0)
            c = plsc.cumsum(mi)
            g_a = p_a + c - 1
            d_act = jnp.where(g_a < K, g_a, g_a + (INB - K))
            d_in = ib_base + p_i + (lanes - c)
            dest = jnp.where(m, d_act, d_in)
            idxv = 16 * i + lanes
            plsc.store_scatter(combo, [dest], idxv)
            c15 = jnp.max(c)
            return p_a + c15, p_i + (16 - c15)

        lax.fori_loop(0, NVREG, pb, (jnp.int32(0), jnp.int32(0)))

        w = combo[pl.ds(3264, 16)]
        combo[pl.ds(3264, 16)] = jnp.where((3264 + lanes) < K, w, 0)
        w2 = combo[pl.ds(32768, 16)]
        combo[pl.ds(32768, 16)] = jnp.where(
            (32768 + lanes) < INB + NBAD, w2, 0)
        combo[pl.ds(32784, 16)] = jnp.zeros((16,), jnp.int32)

        count_region(0, jnp.int32(K), (K + 127) // 128)
        novf = lax.div(t_total - K + 127, jnp.int32(128))
        count_region(INB, t_total - K, novf)

        def vb(j, acc):
            idxv = combo[pl.ds(16 * j, 16)]
            v = plsc.load_gather(xrow, [idxv])
            valbuf[pl.ds(16 * j, 16)] = v
            keep = (16 * j + lanes) < K
            return acc + jnp.where(keep, v, jnp.float32(0.0))

        acc = lax.fori_loop(0, INB // 16, vb,
                            jnp.zeros((16,), jnp.float32))
        tot = jnp.sum(acc)

        def db(j, _):
            valbuf[pl.ds(16 * j, 16)] = valbuf[pl.ds(16 * j, 16)] / tot
            return 0
        lax.fori_loop(0, INB // 16, db, 0)
        pltpu.sync_copy(valbuf, vals_hbm.at[row])

        gather_region(row, 0, GOOD_FULL, GOOD_TAIL, good_hbm, tailg, tgstage)
        gather_region(row, INB, BAD_FULL, BAD_TAIL, bad_hbm, tailb, tbstage)

    plsc.subcore_barrier()

    pltpu.sync_copy(countacc.at[pl.ds(2048 * sid, 2048)], ctmp)
    pltpu.sync_copy(ctmp, cnt_hbm.at[cid, pl.ds(2048 * sid, 2048)])


@jax.jit
def kernel(x, positions, positions_count):
    mesh = plsc.VectorSubcoreMesh(core_axis_name="c", subcore_axis_name="s")
    vals_p, good, bad, cnt = pl.kernel(
        _body,
        out_type=(
            jax.ShapeDtypeStruct((BS, INB), jnp.float32),
            jax.ShapeDtypeStruct((BS, K, D), jnp.float32),
            jax.ShapeDtypeStruct((BS, NBAD, D), jnp.float32),
            jax.ShapeDtypeStruct((NC, N), jnp.float32),
        ),
        mesh=mesh,
        compiler_params=pltpu.CompilerParams(needs_layout_passes=False,
                                             use_tc_tiling_on_sc=False),
        scratch_types=[
            pltpu.VMEM((N,), jnp.float32),
            pltpu.VMEM((COMBO,), jnp.int32),
            pltpu.VMEM((RING, CH, D), jnp.float32),
            pltpu.VMEM((INB,), jnp.float32),
            pltpu.VMEM((2048,), jnp.float32),
            pltpu.VMEM((1, 128), jnp.int32),
            pltpu.VMEM((1, 128), jnp.float32),
            pltpu.VMEM((1, GOOD_TAIL), jnp.int32),
            pltpu.VMEM((1, BAD_TAIL), jnp.int32),
            pltpu.VMEM((GOOD_TAIL, D), jnp.float32),
            pltpu.VMEM((BAD_TAIL, D), jnp.float32),
            pltpu.VMEM_SHARED((N,), jnp.float32),
            pltpu.SemaphoreType.DMA((RING,)),
            pltpu.SemaphoreType.DMA((RING,)),
        ],
    )(x, positions)
    vals_rel = vals_p[:, :K]
    new_count = positions_count + cnt[0] + cnt[1]
    return x, vals_rel, good, bad, new_count


# speedup vs baseline: 3.8929x; 1.0007x over previous
import functools
import math

import jax
import jax.numpy as jnp
from jax import lax
from jax.experimental import pallas as pl
from jax.experimental.layout import Format, Layout
from jax.experimental.pallas import tpu as pltpu
from jax.experimental.pallas import tpu_sc as plsc

BS = 64
N = 32768
D = 16
QUANTILE = 0.9
QI = math.ceil(QUANTILE * (N - 1))
K = N - QI
INB = 3280
NBAD = N - K
COMBO = INB + NBAD + 29

NC = 2
NS = 16
NW = NC * NS
ROWS_PER_W = BS // NW

NVREG = N // 16
CH = 128
RING = 4

LEVELS = ((21, 2048), (10, 2048), (0, 1024))

GOOD_FULL = K // CH
GOOD_TAIL = K - GOOD_FULL * CH
BAD_FULL = NBAD // CH
BAD_TAIL = NBAD - BAD_FULL * CH


def _key16(xv):
    u = plsc.bitcast(xv, jnp.uint32)
    neg = u >= jnp.uint32(0x80000000)
    return jnp.where(neg, ~u, u ^ jnp.uint32(0x80000000))


def _scan_level(ord_ref, nb, kneed):
    nchunks = nb // 16
    lanes = lax.iota(jnp.int32, 16)
    big = jnp.int32(2**30)

    def body(t, carry):
        acc, bmax, smax, amin = carry
        jj = nchunks - 1 - t
        col = ord_ref[pl.ds(0 * nb + 16 * jj, 16)]
        for l in range(1, 16):
            col = col + ord_ref[pl.ds(l * nb + 16 * jj, 16)]
        rev = lax.rev(col, (0,))
        cs = plsc.cumsum(rev)
        a_vec = acc + cs
        bidx = (16 * jj + 15) - lanes
        sel = a_vec >= kneed
        bmax = jnp.maximum(bmax, jnp.max(jnp.where(sel, bidx, -1)))
        smax = jnp.maximum(smax, jnp.max(jnp.where(sel, jnp.int32(-1), a_vec)))
        amin = jnp.minimum(amin, jnp.min(jnp.where(sel, a_vec, big)))
        acc = jnp.max(a_vec)
        return acc, bmax, smax, amin

    acc, bmax, smax, amin = lax.fori_loop(
        0, nchunks,
        body,
        (jnp.int32(0), jnp.int32(-1), jnp.int32(-1), big),
    )
    del acc
    return bmax, jnp.maximum(smax, 0), amin


def _body(x_hbm, pos_hbm, vals_hbm, good_hbm, bad_hbm, cnt_hbm,
          xrow, combo, staging, valbuf, ctmp, bounce, ubuf,
          tailg, tailb, tgstage, tbstage, countacc, gsem, osem):
    cid = lax.axis_index("c")
    sid = lax.axis_index("s")
    wid = sid * NC + cid
    lanes = lax.iota(jnp.int32, 16)
    ones_i = jnp.ones((16,), jnp.int32)

    for ref, base, tailn in ((tailg, GOOD_FULL * CH, GOOD_TAIL),
                             (tailb, BAD_FULL * CH, BAD_TAIL)):
        for off in list(range(0, tailn - 16, 16)) + [tailn - 16]:
            ref[0, pl.ds(off, 16)] = base + off + lanes

    @pl.when(sid == 0)
    def _zero_acc():
        def zbody(i, _):
            xrow[pl.ds(16 * i, 16)] = jnp.zeros((16,), jnp.float32)
            return 0
        lax.fori_loop(0, NVREG, zbody, 0)
        pltpu.sync_copy(xrow, countacc)

    plsc.subcore_barrier()

    def gather_region(row, ord_off, nfull, tailn, out_ref, tail_ref, tstage):
        def wait_g(slot):
            pltpu.make_async_copy(pos_hbm.at[pl.ds(0, CH)], staging.at[slot],
                                  gsem.at[slot]).wait()

        def wait_o(slot):
            pltpu.make_async_copy(staging.at[slot],
                                  out_ref.at[row, pl.ds(0, CH)],
                                  osem.at[slot]).wait()

        def body(g, _):
            slot = lax.rem(g, RING)

            @pl.when(g >= RING)
            def _drain():
                wait_o(slot)

            idx = combo.at[pl.ds(ord_off + CH * g, CH)]
            pltpu.async_copy(pos_hbm.at[idx], staging.at[slot], gsem.at[slot])

            @pl.when(g >= 1)
            def _flush_prev():
                pslot = lax.rem(g - 1, RING)
                wait_g(pslot)
                pltpu.async_copy(staging.at[pslot],
                                 out_ref.at[row, pl.ds(CH * (g - 1), CH)],
                                 osem.at[pslot])

            return 0

        lax.fori_loop(0, nfull, body, 0)
        last = nfull - 1
        lslot = last % RING
        wait_g(lslot)
        pltpu.async_copy(staging.at[lslot],
                         out_ref.at[row, pl.ds(CH * last, CH)],
                         osem.at[lslot])
        tslot = nfull % RING
        wait_o(tslot)
        tpad = (tailn + 7) & ~7
        tidx = combo.at[pl.ds(ord_off + CH * nfull, tpad)]
        pltpu.sync_copy(pos_hbm.at[tidx], staging.at[tslot, pl.ds(0, tpad)])

        def tcopy(r, _):
            tstage[r, pl.ds(0, 16)] = staging[tslot, r, pl.ds(0, 16)]
            return 0
        lax.fori_loop(0, tailn, tcopy, 0)
        pltpu.sync_copy(tstage, out_ref.at[row].at[tail_ref.at[0]])
        for s in range(RING):
            if s != tslot:
                pltpu.make_async_copy(staging.at[s],
                                      out_ref.at[row, pl.ds(0, CH)],
                                      osem.at[s]).wait()

    def count_region(base, limit, ntrips):
        def cbody(cix, _):
            for v in range(8):
                pos = 128 * cix + 16 * v
                raw = combo[pl.ds(base + pos, 16)]
                inr = (pos + lanes) < limit
                bounce[0, pl.ds(16 * v, 16)] = jnp.where(inr, raw, 0)
                ubuf[0, pl.ds(16 * v, 16)] = jnp.where(
                    inr, jnp.float32(1.0), jnp.float32(0.0))
            pltpu.sync_copy(ubuf.at[0], countacc.at[bounce.at[0]], add=True)
            return 0

        lax.fori_loop(0, ntrips, cbody, 0)

    for ridx in range(ROWS_PER_W):
        row = ROWS_PER_W * wid + ridx

        pltpu.sync_copy(x_hbm.at[row], xrow)

        kneed = jnp.int32(K)
        prefix = jnp.uint32(0)
        above = jnp.int32(0)
        t_total = jnp.int32(0)
        pshift = 32
        for (shift, nb) in LEVELS:
            nbits = pshift - shift

            def zb(i, _, nb=nb):
                combo[pl.ds(16 * i, 16)] = jnp.zeros((16,), jnp.int32)
                return 0
            lax.fori_loop(0, nb, zb, 0)

            def hb(i, _, shift=shift, nb=nb, pshift=pshift, prefix=prefix):
                xv = xrow[pl.ds(16 * i, 16)]
                key = _key16(xv)
                bucket = lax.convert_element_type(
                    (key >> shift) & jnp.uint32(nb - 1), jnp.int32)
                flat = (lanes * nb) + bucket
                if pshift == 32:
                    plsc.addupdate_scatter(combo, [flat], ones_i)
                else:
                    match = (key >> pshift) == prefix
                    plsc.addupdate_scatter(combo, [flat], ones_i, mask=match)
                return 0
            lax.fori_loop(0, NVREG, hb, 0)

            b_l, s_l, a_l = _scan_level(combo, nb, kneed)
            t_total = above + a_l
            kneed = kneed - s_l
            above = above + s_l
            prefix = (prefix << nbits) | lax.convert_element_type(
                b_l, jnp.uint32)
            pshift = shift

        q_key = prefix

        ib_base = jnp.int32(INB) + (t_total - K)

        def pb(i, carry):
            p_a, p_i = carry
            xv = xrow[pl.ds(16 * i, 16)]
            key = _key16(xv)
            m = key >= q_key
            mi = jnp.where(m, 1, 0)
            c = plsc.cumsum(mi)
            g_a = p_a + c - 1
            d_act = jnp.where(g_a < K, g_a, g_a + (INB - K))
            d_in = ib_base + p_i + (lanes - c)
            dest = jnp.where(m, d_act, d_in)
            idxv = 16 * i + lanes
            plsc.store_scatter(combo, [dest], idxv)
            c15 = jnp.max(c)
            return p_a + c15, p_i + (16 - c15)

        lax.fori_loop(0, NVREG, pb, (jnp.int32(0), jnp.int32(0)))

        w = combo[pl.ds(3264, 16)]
        combo[pl.ds(3264, 16)] = jnp.where((3264 + lanes) < K, w, 0)
        w2 = combo[pl.ds(32768, 16)]
        combo[pl.ds(32768, 16)] = jnp.where(
            (32768 + lanes) < INB + NBAD, w2, 0)
        combo[pl.ds(32784, 16)] = jnp.zeros((16,), jnp.int32)

        count_region(0, jnp.int32(K), (K + 127) // 128)
        novf = lax.div(t_total - K + 127, jnp.int32(128))
        count_region(INB, t_total - K, novf)

        def vb(j, acc):
            idxv = combo[pl.ds(16 * j, 16)]
            v = plsc.load_gather(xrow, [idxv])
            valbuf[pl.ds(16 * j, 16)] = v
            keep = (16 * j + lanes) < K
            return acc + jnp.where(keep, v, jnp.float32(0.0))

        acc = lax.fori_loop(0, INB // 16, vb,
                            jnp.zeros((16,), jnp.float32))
        tot = jnp.sum(acc)

        def db(j, _):
            valbuf[pl.ds(16 * j, 16)] = valbuf[pl.ds(16 * j, 16)] / tot
            return 0
        lax.fori_loop(0, INB // 16, db, 0)
        pltpu.sync_copy(valbuf, vals_hbm.at[row])

        gather_region(row, 0, GOOD_FULL, GOOD_TAIL, good_hbm, tailg, tgstage)
        gather_region(row, INB, BAD_FULL, BAD_TAIL, bad_hbm, tailb, tbstage)

    plsc.subcore_barrier()

    pltpu.sync_copy(countacc.at[pl.ds(2048 * sid, 2048)], ctmp)
    pltpu.sync_copy(ctmp, cnt_hbm.at[cid, pl.ds(2048 * sid, 2048)])


def _sc_fmt(rank):
    sh = jax.sharding.SingleDeviceSharding(jax.devices()[0])
    return Format(Layout(major_to_minor=tuple(range(rank)), tiling=((8,),)),
                  sh)


_JITTED = None


def kernel(x, positions, positions_count):
    global _JITTED
    if _JITTED is None:
        _JITTED = jax.jit(
            _kernel_impl,
            out_shardings=(None, None, _sc_fmt(3), _sc_fmt(3), None),
        )
    return _JITTED(x, positions, positions_count)


def _kernel_impl(x, positions, positions_count):
    mesh = plsc.VectorSubcoreMesh(core_axis_name="c", subcore_axis_name="s")
    vals_p, good, bad, cnt = pl.kernel(
        _body,
        out_type=(
            jax.ShapeDtypeStruct((BS, INB), jnp.float32),
            jax.ShapeDtypeStruct((BS, K, D), jnp.float32),
            jax.ShapeDtypeStruct((BS, NBAD, D), jnp.float32),
            jax.ShapeDtypeStruct((NC, N), jnp.float32),
        ),
        mesh=mesh,
        compiler_params=pltpu.CompilerParams(needs_layout_passes=False,
                                             use_tc_tiling_on_sc=False),
        scratch_types=[
            pltpu.VMEM((N,), jnp.float32),
            pltpu.VMEM((COMBO,), jnp.int32),
            pltpu.VMEM((RING, CH, D), jnp.float32),
            pltpu.VMEM((INB,), jnp.float32),
            pltpu.VMEM((2048,), jnp.float32),
            pltpu.VMEM((1, 128), jnp.int32),
            pltpu.VMEM((1, 128), jnp.float32),
            pltpu.VMEM((1, GOOD_TAIL), jnp.int32),
            pltpu.VMEM((1, BAD_TAIL), jnp.int32),
            pltpu.VMEM((GOOD_TAIL, D), jnp.float32),
            pltpu.VMEM((BAD_TAIL, D), jnp.float32),
            pltpu.VMEM_SHARED((N,), jnp.float32),
            pltpu.SemaphoreType.DMA((RING,)),
            pltpu.SemaphoreType.DMA((RING,)),
        ],
    )(x, positions)
    vals_rel = vals_p[:, :K]
    new_count = positions_count + cnt[0] + cnt[1]
    return x, vals_rel, good, bad, new_count
